# baseline (speedup 1.0000x reference)
import functools

import jax
import jax.numpy as jnp
from jax import lax
from jax.experimental import pallas as pl
from jax.experimental.pallas import tpu as pltpu
from jax.experimental.pallas import tpu_sc as plsc


@functools.cache
def _sc_fanout(n, t, d, dtype_name):
    dtype = jnp.dtype(dtype_name)
    info = plsc.get_sparse_core_info()
    nc, ns = info.num_cores, info.num_subcores
    nw = nc * ns
    rows_per_w = t // nw
    mesh = plsc.VectorSubcoreMesh(core_axis_name="c", subcore_axis_name="s")

    @functools.partial(
        pl.kernel,
        mesh=mesh,
        out_type=jax.ShapeDtypeStruct((n, t, d), dtype),
        scratch_types=[
            pltpu.VMEM((rows_per_w, d), dtype),
            pltpu.SemaphoreType.DMA,
        ],
    )
    def k(table_hbm, out_hbm, rows_v, sem):
        wid = lax.axis_index("s") * nc + lax.axis_index("c")
        base = wid * rows_per_w
        pltpu.sync_copy(table_hbm.at[pl.ds(base, rows_per_w)], rows_v)
        copies = [
            pltpu.async_copy(rows_v, out_hbm.at[i, pl.ds(base, rows_per_w)], sem)
            for i in range(n)
        ]
        for c in copies:
            c.wait()

    return k


def kernel(inputs, params):
    n, t, d = inputs.shape
    return _sc_fanout(n, t, d, str(params.dtype))(params)
